# baseline (speedup 1.0000x reference)
import jax
import jax.numpy as jnp
from jax import lax
from jax.experimental import pallas as pl
from jax.experimental.pallas import tpu as pltpu

N, D = 50000, 128
BR = 1000
NB = N // BR
NBUF = 10
NT = NB // NBUF


def _tc_body(gu_hbm, gi_hbm, o_ref, ubufs, ibufs, sem_u, sem_i):
    ones = jnp.ones((1, D), jnp.float32)

    def start(b, u):
        pltpu.async_copy(gu_hbm.at[pl.ds(b * BR, BR), :], ubufs.at[u], sem_u.at[u])
        pltpu.async_copy(gi_hbm.at[pl.ds(b * BR, BR), :], ibufs.at[u], sem_i.at[u])

    def wait(u):
        pltpu.make_async_copy(gu_hbm.at[pl.ds(0, BR), :], ubufs.at[u], sem_u.at[u]).wait()
        pltpu.make_async_copy(gi_hbm.at[pl.ds(0, BR), :], ibufs.at[u], sem_i.at[u]).wait()

    for u in range(NBUF):
        start(u, u)

    def outer(t, _):
        for u in range(NBUF):
            b = t * NBUF + u
            wait(u)
            prod = ubufs[u] * ibufs[u]
            o_ref[pl.ds(t * NBUF + u, 1), :] = lax.dot_general(
                ones, prod, (((1,), (1,)), ((), ())),
                preferred_element_type=jnp.float32)

            @pl.when(t < NT - 1)
            def _():
                start(b + NBUF, u)
        return 0

    lax.fori_loop(0, NT, outer, 0)


@jax.jit
def kernel(gu, gi):
    out = pl.pallas_call(
        _tc_body,
        in_specs=[
            pl.BlockSpec(memory_space=pltpu.MemorySpace.HBM),
            pl.BlockSpec(memory_space=pltpu.MemorySpace.HBM),
        ],
        out_shape=jax.ShapeDtypeStruct((NB, BR), jnp.float32),
        scratch_shapes=[
            pltpu.VMEM((NBUF, BR, D), jnp.float32),
            pltpu.VMEM((NBUF, BR, D), jnp.float32),
            pltpu.SemaphoreType.DMA((NBUF,)),
            pltpu.SemaphoreType.DMA((NBUF,)),
        ],
    )(gu, gi)
    return out.reshape(N)


# speedup vs baseline: 1.0459x; 1.0459x over previous
import jax
import jax.numpy as jnp
from jax import lax
from jax.experimental import pallas as pl
from jax.experimental.pallas import tpu as pltpu

N, D = 50000, 128
BR = 2000
NB = N // BR
NBUF = 5
NT = NB // NBUF


def _tc_body(gu_hbm, gi_hbm, o_ref, ubufs, ibufs, sem_u, sem_i):
    ones = jnp.ones((1, D), jnp.float32)

    def start(b, u):
        pltpu.async_copy(gu_hbm.at[pl.ds(b * BR, BR), :], ubufs.at[u], sem_u.at[u])
        pltpu.async_copy(gi_hbm.at[pl.ds(b * BR, BR), :], ibufs.at[u], sem_i.at[u])

    def wait(u):
        pltpu.make_async_copy(gu_hbm.at[pl.ds(0, BR), :], ubufs.at[u], sem_u.at[u]).wait()
        pltpu.make_async_copy(gi_hbm.at[pl.ds(0, BR), :], ibufs.at[u], sem_i.at[u]).wait()

    for u in range(NBUF):
        start(u, u)

    def outer(t, _):
        for u in range(NBUF):
            b = t * NBUF + u
            wait(u)
            prod = ubufs[u] * ibufs[u]
            o_ref[pl.ds(t * NBUF + u, 1), :] = lax.dot_general(
                ones, prod, (((1,), (1,)), ((), ())),
                preferred_element_type=jnp.float32)

            @pl.when(t < NT - 1)
            def _():
                start(b + NBUF, u)
        return 0

    lax.fori_loop(0, NT, outer, 0)


@jax.jit
def kernel(gu, gi):
    out = pl.pallas_call(
        _tc_body,
        in_specs=[
            pl.BlockSpec(memory_space=pltpu.MemorySpace.HBM),
            pl.BlockSpec(memory_space=pltpu.MemorySpace.HBM),
        ],
        out_shape=jax.ShapeDtypeStruct((NB, BR), jnp.float32),
        scratch_shapes=[
            pltpu.VMEM((NBUF, BR, D), jnp.float32),
            pltpu.VMEM((NBUF, BR, D), jnp.float32),
            pltpu.SemaphoreType.DMA((NBUF,)),
            pltpu.SemaphoreType.DMA((NBUF,)),
        ],
    )(gu, gi)
    return out.reshape(N)
